# baseline (speedup 1.0000x reference)
import functools
import math

import jax
import jax.numpy as jnp
from jax import lax
from jax.experimental import pallas as pl
from jax.experimental.pallas import tpu as pltpu
from jax.experimental.pallas import tpu_sc as plsc

B = 16384
EMB_DIM = 64
OUT_DIM = 128
NUM_AGE = 5
NUM_GENDER = 2
NUM_ETH = 7
NUM_COMBO = NUM_AGE * NUM_GENDER * NUM_ETH

NC = 2
NS = 16
LANES = 16
NW = NC * NS
BPW = B // NW
CHUNK = 64
NCHUNK = BPW // CHUNK


def _lut_body(ea_ref, eg_ref, ee_ref, w_ref, b_ref, lut_ref):
    scale = jnp.float32(1.0 / math.sqrt(EMB_DIM * 3))
    w = w_ref[...] * scale
    dn = (((1,), (1,)), ((), ()))
    pa = lax.dot_general(ea_ref[...], w[:, 0:EMB_DIM], dn,
                         precision=lax.Precision.HIGHEST,
                         preferred_element_type=jnp.float32)
    pg = lax.dot_general(eg_ref[...], w[:, EMB_DIM:2 * EMB_DIM], dn,
                         precision=lax.Precision.HIGHEST,
                         preferred_element_type=jnp.float32)
    pe = lax.dot_general(ee_ref[...], w[:, 2 * EMB_DIM:3 * EMB_DIM], dn,
                         precision=lax.Precision.HIGHEST,
                         preferred_element_type=jnp.float32)

    def onehot(vals, n):
        j = lax.broadcasted_iota(jnp.int32, (NUM_COMBO, n), 1)
        return (j == vals).astype(jnp.float32)

    i_a = lax.broadcasted_iota(jnp.int32, (NUM_COMBO, NUM_AGE), 0)
    i_g = lax.broadcasted_iota(jnp.int32, (NUM_COMBO, NUM_GENDER), 0)
    i_e = lax.broadcasted_iota(jnp.int32, (NUM_COMBO, NUM_ETH), 0)
    sel_a = onehot(i_a // (NUM_GENDER * NUM_ETH), NUM_AGE)
    sel_g = onehot((i_g // NUM_ETH) % NUM_GENDER, NUM_GENDER)
    sel_e = onehot(i_e % NUM_ETH, NUM_ETH)

    dn2 = (((1,), (0,)), ((), ()))
    acc = lax.dot_general(sel_a, pa, dn2,
                          precision=lax.Precision.HIGHEST,
                          preferred_element_type=jnp.float32)
    acc = acc + lax.dot_general(sel_g, pg, dn2,
                                precision=lax.Precision.HIGHEST,
                                preferred_element_type=jnp.float32)
    acc = acc + lax.dot_general(sel_e, pe, dn2,
                                precision=lax.Precision.HIGHEST,
                                preferred_element_type=jnp.float32)
    acc = acc + b_ref[...]
    lut_ref[...] = jnp.where(acc >= 0, acc, jnp.float32(0.2) * acc)


def _build_lut(emb_age, emb_gender, emb_eth, W, b):
    return pl.pallas_call(
        _lut_body,
        out_shape=jax.ShapeDtypeStruct((NUM_COMBO, OUT_DIM), jnp.float32),
    )(emb_age, emb_gender, emb_eth, W, b.reshape(1, OUT_DIM))


def _gather_body(lut_hbm, age_hbm, gender_hbm, eth_hbm, out_hbm,
                 age_v, gen_v, eth_v, idx_v, rows_v, lut_s,
                 sem_in, sem_g, sem_out):
    cid = lax.axis_index("c")
    sid = lax.axis_index("s")
    wid = sid * NC + cid
    base = wid * BPW
    in_cp = [
        pltpu.async_copy(age_hbm.at[pl.ds(base, BPW)], age_v, sem_in),
        pltpu.async_copy(gender_hbm.at[pl.ds(base, BPW)], gen_v, sem_in),
        pltpu.async_copy(eth_hbm.at[pl.ds(base, BPW)], eth_v, sem_in),
    ]

    @pl.when(sid == 0)
    def _():
        pltpu.sync_copy(lut_hbm, lut_s)

    plsc.subcore_barrier()

    for cp in in_cp:
        cp.wait()

    kg = jnp.int32(NUM_GENDER * NUM_ETH)
    ke = jnp.int32(NUM_ETH)
    per_chunk = CHUNK // LANES
    gathers = []
    out_cp = []
    for j in range(NCHUNK):
        for o in range(per_chunk):
            k = j * per_chunk + o
            s = pl.ds(k * LANES, LANES)
            v = age_v[s] * kg + gen_v[s] * ke + eth_v[s]
            idx_v[j, pl.ds(o * LANES, LANES)] = v
        gathers.append(pltpu.async_copy(
            lut_s.at[idx_v.at[j]],
            rows_v.at[pl.ds(j * CHUNK, CHUNK)], sem_g.at[j]))
        if j > 0:
            gathers[j - 1].wait()
            out_cp.append(pltpu.async_copy(
                rows_v.at[pl.ds((j - 1) * CHUNK, CHUNK)],
                out_hbm.at[pl.ds(base + (j - 1) * CHUNK, CHUNK)], sem_out))
    gathers[NCHUNK - 1].wait()
    out_cp.append(pltpu.async_copy(
        rows_v.at[pl.ds((NCHUNK - 1) * CHUNK, CHUNK)],
        out_hbm.at[pl.ds(base + (NCHUNK - 1) * CHUNK, CHUNK)], sem_out))
    for cp in out_cp:
        cp.wait()


@functools.cache
def _gather():
    return pl.kernel(
        _gather_body,
        out_type=jax.ShapeDtypeStruct((B, OUT_DIM), jnp.float32),
        mesh=plsc.VectorSubcoreMesh(core_axis_name="c", subcore_axis_name="s",
                                    num_cores=NC, num_subcores=NS),
        compiler_params=pltpu.CompilerParams(
            needs_layout_passes=False,
            disable_bounds_checks=True,
            disable_semaphore_checks=True,
        ),
        scratch_types=[
            pltpu.VMEM((BPW,), jnp.int32),
            pltpu.VMEM((BPW,), jnp.int32),
            pltpu.VMEM((BPW,), jnp.int32),
            pltpu.VMEM((NCHUNK, CHUNK), jnp.int32),
            pltpu.VMEM((BPW, OUT_DIM), jnp.float32),
            pltpu.VMEM_SHARED((NUM_COMBO, OUT_DIM), jnp.float32),
            pltpu.SemaphoreType.DMA,
            pltpu.SemaphoreType.DMA((NCHUNK,)),
            pltpu.SemaphoreType.DMA,
        ],
    )


@jax.jit
def kernel(age, gender, eth, emb_age, emb_gender, emb_eth, W, b):
    lut = _build_lut(emb_age, emb_gender, emb_eth, W, b)
    return _gather()(lut, age.astype(jnp.int32), gender.astype(jnp.int32),
                     eth.astype(jnp.int32))


# speedup vs baseline: 1.0140x; 1.0140x over previous
import functools
import math

import jax
import jax.numpy as jnp
from jax import lax
from jax.experimental import pallas as pl
from jax.experimental.pallas import tpu as pltpu
from jax.experimental.pallas import tpu_sc as plsc

B = 16384
EMB_DIM = 64
OUT_DIM = 128
NUM_AGE = 5
NUM_GENDER = 2
NUM_ETH = 7
NUM_COMBO = NUM_AGE * NUM_GENDER * NUM_ETH

NC = 2
NS = 16
LANES = 16
NW = NC * NS
BPW = B // NW
CHUNK = 64
NCHUNK = BPW // CHUNK


def _lut_body(ea_ref, eg_ref, ee_ref, w_ref, b_ref, lut_ref):
    scale = jnp.float32(1.0 / math.sqrt(EMB_DIM * 3))
    w = w_ref[...] * scale
    dn = (((1,), (1,)), ((), ()))
    pa = lax.dot_general(ea_ref[...], w[:, 0:EMB_DIM], dn,
                         precision=lax.Precision.HIGHEST,
                         preferred_element_type=jnp.float32)
    pg = lax.dot_general(eg_ref[...], w[:, EMB_DIM:2 * EMB_DIM], dn,
                         precision=lax.Precision.HIGHEST,
                         preferred_element_type=jnp.float32)
    pe = lax.dot_general(ee_ref[...], w[:, 2 * EMB_DIM:3 * EMB_DIM], dn,
                         precision=lax.Precision.HIGHEST,
                         preferred_element_type=jnp.float32)

    def onehot(vals, n):
        j = lax.broadcasted_iota(jnp.int32, (NUM_COMBO, n), 1)
        return (j == vals).astype(jnp.float32)

    i_a = lax.broadcasted_iota(jnp.int32, (NUM_COMBO, NUM_AGE), 0)
    i_g = lax.broadcasted_iota(jnp.int32, (NUM_COMBO, NUM_GENDER), 0)
    i_e = lax.broadcasted_iota(jnp.int32, (NUM_COMBO, NUM_ETH), 0)
    sel_a = onehot(i_a // (NUM_GENDER * NUM_ETH), NUM_AGE)
    sel_g = onehot((i_g // NUM_ETH) % NUM_GENDER, NUM_GENDER)
    sel_e = onehot(i_e % NUM_ETH, NUM_ETH)

    dn2 = (((1,), (0,)), ((), ()))
    acc = lax.dot_general(sel_a, pa, dn2,
                          precision=lax.Precision.HIGHEST,
                          preferred_element_type=jnp.float32)
    acc = acc + lax.dot_general(sel_g, pg, dn2,
                                precision=lax.Precision.HIGHEST,
                                preferred_element_type=jnp.float32)
    acc = acc + lax.dot_general(sel_e, pe, dn2,
                                precision=lax.Precision.HIGHEST,
                                preferred_element_type=jnp.float32)
    acc = acc + b_ref[...]
    lut_ref[...] = jnp.where(acc >= 0, acc, jnp.float32(0.2) * acc)


def _build_lut(emb_age, emb_gender, emb_eth, W, b):
    return pl.pallas_call(
        _lut_body,
        out_shape=jax.ShapeDtypeStruct((NUM_COMBO, OUT_DIM), jnp.float32),
    )(emb_age, emb_gender, emb_eth, W, b.reshape(1, OUT_DIM))


def _gather_body(lut_hbm, age_hbm, gender_hbm, eth_hbm, out_hbm,
                 age_v, gen_v, eth_v, idx_v, rows_v, lut_s,
                 sem_in, sem_g, sem_out):
    cid = lax.axis_index("c")
    sid = lax.axis_index("s")
    wid = sid * NC + cid
    base = wid * BPW
    in_cp = [
        pltpu.async_copy(age_hbm.at[pl.ds(base, BPW)], age_v, sem_in),
        pltpu.async_copy(gender_hbm.at[pl.ds(base, BPW)], gen_v, sem_in),
        pltpu.async_copy(eth_hbm.at[pl.ds(base, BPW)], eth_v, sem_in),
    ]

    @pl.when(sid == 0)
    def _():
        pltpu.sync_copy(lut_hbm, lut_s)

    for cp in in_cp:
        cp.wait()

    kg = jnp.int32(NUM_GENDER * NUM_ETH)
    ke = jnp.int32(NUM_ETH)
    per_chunk = CHUNK // LANES
    for k in range(BPW // LANES):
        s = pl.ds(k * LANES, LANES)
        v = age_v[s] * kg + gen_v[s] * ke + eth_v[s]
        idx_v[k // per_chunk, pl.ds((k % per_chunk) * LANES, LANES)] = v

    plsc.subcore_barrier()

    gathers = [
        pltpu.async_copy(lut_s.at[idx_v.at[j]],
                         rows_v.at[pl.ds(j * CHUNK, CHUNK)], sem_g.at[j])
        for j in range(NCHUNK)
    ]
    out_cp = []
    for j in range(NCHUNK):
        gathers[j].wait()
        out_cp.append(pltpu.async_copy(
            rows_v.at[pl.ds(j * CHUNK, CHUNK)],
            out_hbm.at[pl.ds(base + j * CHUNK, CHUNK)], sem_out))
    for cp in out_cp:
        cp.wait()


@functools.cache
def _gather():
    return pl.kernel(
        _gather_body,
        out_type=jax.ShapeDtypeStruct((B, OUT_DIM), jnp.float32),
        mesh=plsc.VectorSubcoreMesh(core_axis_name="c", subcore_axis_name="s",
                                    num_cores=NC, num_subcores=NS),
        compiler_params=pltpu.CompilerParams(
            needs_layout_passes=False,
            disable_bounds_checks=True,
            disable_semaphore_checks=True,
        ),
        scratch_types=[
            pltpu.VMEM((BPW,), jnp.int32),
            pltpu.VMEM((BPW,), jnp.int32),
            pltpu.VMEM((BPW,), jnp.int32),
            pltpu.VMEM((NCHUNK, CHUNK), jnp.int32),
            pltpu.VMEM((BPW, OUT_DIM), jnp.float32),
            pltpu.VMEM_SHARED((NUM_COMBO, OUT_DIM), jnp.float32),
            pltpu.SemaphoreType.DMA,
            pltpu.SemaphoreType.DMA((NCHUNK,)),
            pltpu.SemaphoreType.DMA,
        ],
    )


@jax.jit
def kernel(age, gender, eth, emb_age, emb_gender, emb_eth, W, b):
    lut = _build_lut(emb_age, emb_gender, emb_eth, W, b)
    return _gather()(lut, age.astype(jnp.int32), gender.astype(jnp.int32),
                     eth.astype(jnp.int32))


# speedup vs baseline: 1.0209x; 1.0068x over previous
import functools
import math

import jax
import jax.numpy as jnp
from jax import lax
from jax.experimental import pallas as pl
from jax.experimental.pallas import tpu as pltpu
from jax.experimental.pallas import tpu_sc as plsc

B = 16384
EMB_DIM = 64
OUT_DIM = 128
NUM_AGE = 5
NUM_GENDER = 2
NUM_ETH = 7
NUM_COMBO = NUM_AGE * NUM_GENDER * NUM_ETH

NC = 2
NS = 16
LANES = 16
NW = NC * NS
BPW = B // NW
CHUNK = 32
NCHUNK = BPW // CHUNK


def _lut_body(ea_ref, eg_ref, ee_ref, w_ref, b_ref, lut_ref):
    scale = jnp.float32(1.0 / math.sqrt(EMB_DIM * 3))
    w = w_ref[...] * scale
    dn = (((1,), (1,)), ((), ()))
    pa = lax.dot_general(ea_ref[...], w[:, 0:EMB_DIM], dn,
                         precision=lax.Precision.HIGHEST,
                         preferred_element_type=jnp.float32)
    pg = lax.dot_general(eg_ref[...], w[:, EMB_DIM:2 * EMB_DIM], dn,
                         precision=lax.Precision.HIGHEST,
                         preferred_element_type=jnp.float32)
    pe = lax.dot_general(ee_ref[...], w[:, 2 * EMB_DIM:3 * EMB_DIM], dn,
                         precision=lax.Precision.HIGHEST,
                         preferred_element_type=jnp.float32)

    def onehot(vals, n):
        j = lax.broadcasted_iota(jnp.int32, (NUM_COMBO, n), 1)
        return (j == vals).astype(jnp.float32)

    i_a = lax.broadcasted_iota(jnp.int32, (NUM_COMBO, NUM_AGE), 0)
    i_g = lax.broadcasted_iota(jnp.int32, (NUM_COMBO, NUM_GENDER), 0)
    i_e = lax.broadcasted_iota(jnp.int32, (NUM_COMBO, NUM_ETH), 0)
    sel_a = onehot(i_a // (NUM_GENDER * NUM_ETH), NUM_AGE)
    sel_g = onehot((i_g // NUM_ETH) % NUM_GENDER, NUM_GENDER)
    sel_e = onehot(i_e % NUM_ETH, NUM_ETH)

    dn2 = (((1,), (0,)), ((), ()))
    acc = lax.dot_general(sel_a, pa, dn2,
                          precision=lax.Precision.HIGHEST,
                          preferred_element_type=jnp.float32)
    acc = acc + lax.dot_general(sel_g, pg, dn2,
                                precision=lax.Precision.HIGHEST,
                                preferred_element_type=jnp.float32)
    acc = acc + lax.dot_general(sel_e, pe, dn2,
                                precision=lax.Precision.HIGHEST,
                                preferred_element_type=jnp.float32)
    acc = acc + b_ref[...]
    lut_ref[...] = jnp.where(acc >= 0, acc, jnp.float32(0.2) * acc)


def _build_lut(emb_age, emb_gender, emb_eth, W, b):
    return pl.pallas_call(
        _lut_body,
        out_shape=jax.ShapeDtypeStruct((NUM_COMBO, OUT_DIM), jnp.float32),
    )(emb_age, emb_gender, emb_eth, W, b.reshape(1, OUT_DIM))


def _gather_body(lut_hbm, age_hbm, gender_hbm, eth_hbm, out_hbm,
                 age_v, gen_v, eth_v, idx_v, rows_v, lut_s,
                 sem_in, sem_g, sem_out):
    cid = lax.axis_index("c")
    sid = lax.axis_index("s")
    wid = sid * NC + cid
    base = wid * BPW
    in_cp = [
        pltpu.async_copy(age_hbm.at[pl.ds(base, BPW)], age_v, sem_in),
        pltpu.async_copy(gender_hbm.at[pl.ds(base, BPW)], gen_v, sem_in),
        pltpu.async_copy(eth_hbm.at[pl.ds(base, BPW)], eth_v, sem_in),
    ]

    @pl.when(sid == 0)
    def _():
        pltpu.sync_copy(lut_hbm, lut_s)

    for cp in in_cp:
        cp.wait()

    kg = jnp.int32(NUM_GENDER * NUM_ETH)
    ke = jnp.int32(NUM_ETH)
    per_chunk = CHUNK // LANES
    for k in range(BPW // LANES):
        s = pl.ds(k * LANES, LANES)
        v = age_v[s] * kg + gen_v[s] * ke + eth_v[s]
        idx_v[k // per_chunk, pl.ds((k % per_chunk) * LANES, LANES)] = v

    plsc.subcore_barrier()

    gathers = [
        pltpu.async_copy(lut_s.at[idx_v.at[j]],
                         rows_v.at[pl.ds(j * CHUNK, CHUNK)], sem_g.at[j])
        for j in range(NCHUNK)
    ]
    out_cp = []
    for j in range(NCHUNK):
        gathers[j].wait()
        out_cp.append(pltpu.async_copy(
            rows_v.at[pl.ds(j * CHUNK, CHUNK)],
            out_hbm.at[pl.ds(base + j * CHUNK, CHUNK)], sem_out))
    for cp in out_cp:
        cp.wait()


@functools.cache
def _gather():
    return pl.kernel(
        _gather_body,
        out_type=jax.ShapeDtypeStruct((B, OUT_DIM), jnp.float32),
        mesh=plsc.VectorSubcoreMesh(core_axis_name="c", subcore_axis_name="s",
                                    num_cores=NC, num_subcores=NS),
        compiler_params=pltpu.CompilerParams(
            needs_layout_passes=False,
            disable_bounds_checks=True,
            disable_semaphore_checks=True,
        ),
        scratch_types=[
            pltpu.VMEM((BPW,), jnp.int32),
            pltpu.VMEM((BPW,), jnp.int32),
            pltpu.VMEM((BPW,), jnp.int32),
            pltpu.VMEM((NCHUNK, CHUNK), jnp.int32),
            pltpu.VMEM((BPW, OUT_DIM), jnp.float32),
            pltpu.VMEM_SHARED((NUM_COMBO, OUT_DIM), jnp.float32),
            pltpu.SemaphoreType.DMA,
            pltpu.SemaphoreType.DMA((NCHUNK,)),
            pltpu.SemaphoreType.DMA,
        ],
    )


@jax.jit
def kernel(age, gender, eth, emb_age, emb_gender, emb_eth, W, b):
    lut = _build_lut(emb_age, emb_gender, emb_eth, W, b)
    return _gather()(lut, age.astype(jnp.int32), gender.astype(jnp.int32),
                     eth.astype(jnp.int32))
